# Initial kernel scaffold; baseline (speedup 1.0000x reference)
import jax
import jax.numpy as jnp
from jax.experimental import pallas as pl


def kernel(x, edge_index, W1, b1, gamma1, beta1, W2, b2, gamma2, beta2, W3, b3):
    raise NotImplementedError("write your pallas kernel here")


# speedup vs baseline: 4.8105x; 4.8105x over previous
import functools

import jax
import jax.numpy as jnp
from jax import lax
from jax.experimental import pallas as pl
from jax.experimental.pallas import tpu as pltpu
from jax.experimental.pallas import tpu_sc as plsc

N = 10000
NP = 10240
E = 160000
CH = 128
NCHUNK_PAD = 1280
EPAD = NCHUNK_PAD * CH
ROWS_PER_TILE = NP // 16

_MESH = plsc.VectorSubcoreMesh(
    core_axis_name="c", subcore_axis_name="s", num_cores=2, num_subcores=16)


@functools.partial(
    pl.kernel,
    out_type=jax.ShapeDtypeStruct((2, NP, 128), jnp.float32),
    mesh=_MESH,
    scratch_types=[
        pltpu.VMEM((80, CH), jnp.int32),
        pltpu.VMEM((CH, 128), jnp.float32),
        pltpu.VMEM_SHARED((NP + 1, 128), jnp.float32),
    ],
)
def _sc_hist(edges, ones_h, zeros_h, out, idx_loc, ones_v, acc):
    c = lax.axis_index("c")
    s = lax.axis_index("s")
    r0 = s * ROWS_PER_TILE
    pltpu.sync_copy(zeros_h.at[pl.ds(r0, ROWS_PER_TILE)],
                    acc.at[pl.ds(r0, ROWS_PER_TILE)])
    pltpu.sync_copy(ones_h, ones_v)
    pltpu.sync_copy(edges.at[c, pl.ds(s * 80, 80)], idx_loc)
    plsc.subcore_barrier()

    def chunk(j, carry):
        pltpu.sync_copy(ones_v, acc.at[idx_loc.at[j]], add=True)
        return carry

    lax.fori_loop(0, 80, chunk, 0)
    plsc.subcore_barrier()
    pltpu.sync_copy(acc.at[pl.ds(r0, ROWS_PER_TILE)],
                    out.at[c, pl.ds(r0, ROWS_PER_TILE)])


@functools.partial(
    pl.kernel,
    out_type=jax.ShapeDtypeStruct((2, NP, 128), jnp.float32),
    mesh=_MESH,
    scratch_types=[
        pltpu.VMEM((80, CH), jnp.int32),
        pltpu.VMEM((80, CH), jnp.int32),
        pltpu.VMEM((CH, 128), jnp.float32),
        pltpu.SemaphoreType.DMA,
        pltpu.VMEM_SHARED((NP + 1, 128), jnp.float32),
    ],
)
def _sc_spmm256(pflat, src2, dstp, zeros_h, out, src_loc, dst_loc, rows, sem,
                acc):
    c = lax.axis_index("c")
    s = lax.axis_index("s")
    r0 = s * ROWS_PER_TILE
    pltpu.sync_copy(zeros_h.at[pl.ds(r0, ROWS_PER_TILE)],
                    acc.at[pl.ds(r0, ROWS_PER_TILE)])
    pltpu.sync_copy(src2.at[c, pl.ds(s * 80, 80)], src_loc)
    pltpu.sync_copy(dstp.at[pl.ds(s * 80, 80)], dst_loc)
    plsc.subcore_barrier()

    def chunk(j, carry):
        pltpu.async_copy(pflat.at[src_loc.at[j]], rows, sem).wait()
        pltpu.sync_copy(rows, acc.at[dst_loc.at[j]], add=True)
        return carry

    lax.fori_loop(0, 80, chunk, 0)
    plsc.subcore_barrier()
    pltpu.sync_copy(acc.at[pl.ds(r0, ROWS_PER_TILE)],
                    out.at[c, pl.ds(r0, ROWS_PER_TILE)])


@functools.partial(
    pl.kernel,
    out_type=jax.ShapeDtypeStruct((2, NP, 128), jnp.float32),
    mesh=_MESH,
    scratch_types=[
        pltpu.VMEM((40, CH), jnp.int32),
        pltpu.VMEM((40, CH), jnp.int32),
        pltpu.VMEM((CH, 128), jnp.float32),
        pltpu.SemaphoreType.DMA,
        pltpu.VMEM_SHARED((NP + 1, 128), jnp.float32),
    ],
)
def _sc_spmm64(table, srcp, dstp, zeros_h, out, src_loc, dst_loc, rows, sem,
               acc):
    c = lax.axis_index("c")
    s = lax.axis_index("s")
    r0 = s * ROWS_PER_TILE
    base = c * 640 + s * 40
    pltpu.sync_copy(zeros_h.at[pl.ds(r0, ROWS_PER_TILE)],
                    acc.at[pl.ds(r0, ROWS_PER_TILE)])
    pltpu.sync_copy(srcp.at[pl.ds(base, 40)], src_loc)
    pltpu.sync_copy(dstp.at[pl.ds(base, 40)], dst_loc)
    plsc.subcore_barrier()

    def chunk(j, carry):
        pltpu.async_copy(table.at[src_loc.at[j]], rows, sem).wait()
        pltpu.sync_copy(rows, acc.at[dst_loc.at[j]], add=True)
        return carry

    lax.fori_loop(0, 40, chunk, 0)
    plsc.subcore_barrier()
    pltpu.sync_copy(acc.at[pl.ds(r0, ROWS_PER_TILE)],
                    out.at[c, pl.ds(r0, ROWS_PER_TILE)])


@functools.partial(
    pl.kernel,
    out_type=jax.ShapeDtypeStruct((E, 128), jnp.float32),
    mesh=_MESH,
    scratch_types=[
        pltpu.VMEM((40, CH), jnp.int32),
        pltpu.VMEM((CH, 128), jnp.float32),
        pltpu.SemaphoreType.DMA,
    ],
)
def _sc_gather64(table, srcp, out, src_loc, rows, sem):
    c = lax.axis_index("c")
    s = lax.axis_index("s")
    base = c * 640 + s * 40
    pltpu.sync_copy(srcp.at[pl.ds(base, 40)], src_loc)

    def chunk(j, carry):
        ch = base + j

        @pl.when(ch < E // CH)
        def _():
            pltpu.async_copy(table.at[src_loc.at[j]], rows, sem).wait()
            pltpu.sync_copy(rows, out.at[pl.ds(ch * CH, CH)])

        return carry

    lax.fori_loop(0, 40, chunk, 0)


def _tc_aux(deg16):
    def body(d_ref, o_ref):
        dout = d_ref[0][:, :16]
        din = d_ref[1][:, :16]
        a = lax.rsqrt(jnp.maximum(dout, 1.0))
        b = lax.rsqrt(jnp.maximum(din, 1.0))
        o_ref[0] = a
        o_ref[1] = b
        o_ref[2] = a * b
        o_ref[3] = din

    return pl.pallas_call(
        body, out_shape=jax.ShapeDtypeStruct((4, NP, 16), jnp.float32),
    )(deg16)


def _tc_lin1(x, W1, b1r, a16):
    R = 512

    def body(x_ref, w_ref, b_ref, a_ref, o_ref):
        h = jnp.dot(x_ref[...], w_ref[...],
                    preferred_element_type=jnp.float32) + b_ref[...]
        ph = h * a_ref[:, 0:1]
        o_ref[0] = ph[:, :128]
        o_ref[1] = ph[:, 128:]

    return pl.pallas_call(
        body,
        grid=(NP // R,),
        in_specs=[
            pl.BlockSpec((R, 128), lambda r: (r, 0)),
            pl.BlockSpec((128, 256), lambda r: (0, 0)),
            pl.BlockSpec((1, 256), lambda r: (0, 0)),
            pl.BlockSpec((R, 16), lambda r: (r, 0)),
        ],
        out_specs=pl.BlockSpec((2, R, 128), lambda r: (0, r, 0)),
        out_shape=jax.ShapeDtypeStruct((2, NP, 128), jnp.float32),
    )(x, W1, b1r, a16)


def _tc_bn_lin(agg, b16, g_r, be_r, W2, b2r, a16):
    R = 512

    def body(agg_ref, b_ref, g_ref, be_ref, w_ref, b2_ref, a_ref, o_ref,
             st_ref):
        p = pl.program_id(0)
        r = pl.program_id(1)
        o = jnp.concatenate([agg_ref[0], agg_ref[1]], axis=1) * b_ref[:, 0:1]

        @pl.when(jnp.logical_and(p == 0, r == 0))
        def _():
            st_ref[...] = jnp.zeros_like(st_ref)

        @pl.when(p == 0)
        def _():
            st_ref[0:1, :] += jnp.sum(o, axis=0, keepdims=True)
            st_ref[1:2, :] += jnp.sum(o * o, axis=0, keepdims=True)

        @pl.when(p == 1)
        def _():
            mu = st_ref[0:1, :] / float(N)
            var = st_ref[1:2, :] / float(N) - mu * mu
            z = (o - mu) * lax.rsqrt(var + 1e-5) * g_ref[...] + be_ref[...]
            z = jnp.maximum(z, 0.0)
            ph = (jnp.dot(z, w_ref[...], preferred_element_type=jnp.float32)
                  + b2_ref[...]) * a_ref[:, 0:1]
            o_ref[0] = ph[:, :128]
            o_ref[1] = ph[:, 128:]

    return pl.pallas_call(
        body,
        grid=(2, NP // R),
        in_specs=[
            pl.BlockSpec((2, R, 128), lambda p, r: (0, r, 0)),
            pl.BlockSpec((R, 16), lambda p, r: (r, 0)),
            pl.BlockSpec((1, 256), lambda p, r: (0, 0)),
            pl.BlockSpec((1, 256), lambda p, r: (0, 0)),
            pl.BlockSpec((256, 256), lambda p, r: (0, 0)),
            pl.BlockSpec((1, 256), lambda p, r: (0, 0)),
            pl.BlockSpec((R, 16), lambda p, r: (r, 0)),
        ],
        out_specs=pl.BlockSpec((2, R, 128), lambda p, r: (0, r, 0)),
        out_shape=jax.ShapeDtypeStruct((2, NP, 128), jnp.float32),
        scratch_shapes=[pltpu.VMEM((2, 256), jnp.float32)],
    )(agg, b16, g_r, be_r, W2, b2r, a16)


def _tc_bn_lin3(agg, b16, g_r, be_r, W3a, W3b, b3r, ind16):
    R = 512

    def body(agg_ref, b_ref, g_ref, be_ref, wa_ref, wb_ref, b3_ref, ind_ref,
             o_ref, st_ref):
        p = pl.program_id(0)
        r = pl.program_id(1)
        o = jnp.concatenate([agg_ref[0], agg_ref[1]], axis=1) * b_ref[:, 0:1]

        @pl.when(jnp.logical_and(p == 0, r == 0))
        def _():
            st_ref[...] = jnp.zeros_like(st_ref)

        @pl.when(p == 0)
        def _():
            st_ref[0:1, :] += jnp.sum(o, axis=0, keepdims=True)
            st_ref[1:2, :] += jnp.sum(o * o, axis=0, keepdims=True)

        @pl.when(p == 1)
        def _():
            mu = st_ref[0:1, :] / float(N)
            var = st_ref[1:2, :] / float(N) - mu * mu
            z = (o - mu) * lax.rsqrt(var + 1e-5) * g_ref[...] + be_ref[...]
            z = jnp.maximum(z, 0.0)
            ha = jnp.dot(z, wa_ref[...], preferred_element_type=jnp.float32)
            init = ind_ref[:, 0:1] * (
                jnp.dot(z, wb_ref[...], preferred_element_type=jnp.float32)
                + b3_ref[...])
            o_ref[...] = jnp.concatenate([ha, init], axis=1)

    return pl.pallas_call(
        body,
        grid=(2, NP // R),
        in_specs=[
            pl.BlockSpec((2, R, 128), lambda p, r: (0, r, 0)),
            pl.BlockSpec((R, 16), lambda p, r: (r, 0)),
            pl.BlockSpec((1, 256), lambda p, r: (0, 0)),
            pl.BlockSpec((1, 256), lambda p, r: (0, 0)),
            pl.BlockSpec((256, 64), lambda p, r: (0, 0)),
            pl.BlockSpec((256, 64), lambda p, r: (0, 0)),
            pl.BlockSpec((1, 64), lambda p, r: (0, 0)),
            pl.BlockSpec((R, 16), lambda p, r: (r, 0)),
        ],
        out_specs=pl.BlockSpec((R, 128), lambda p, r: (r, 0)),
        out_shape=jax.ShapeDtypeStruct((NP, 128), jnp.float32),
        scratch_shapes=[pltpu.VMEM((2, 256), jnp.float32)],
    )(agg, b16, g_r, be_r, W3a, W3b, b3r, ind16)


def _tc_final(aggD, hainit, nn16):
    def body(agg_ref, hi_ref, nn_ref, o_ref):
        left = nn_ref[:, 0:1] * (agg_ref[0][:, :64] + agg_ref[1][:, :64]
                                 + hi_ref[:, 64:])
        o_ref[...] = jnp.concatenate([left, jnp.zeros_like(left)], axis=1)

    return pl.pallas_call(
        body, out_shape=jax.ShapeDtypeStruct((NP, 128), jnp.float32),
    )(aggD, hainit, nn16)


def kernel(x, edge_index, W1, b1, gamma1, beta1, W2, b2, gamma2, beta2, W3,
           b3):
    src = edge_index[0]
    dst = edge_index[1]
    npad = EPAD - E
    src0 = jnp.concatenate(
        [src, jnp.zeros((npad,), jnp.int32)]).reshape(NCHUNK_PAD, CH)
    srcD = jnp.concatenate(
        [src, jnp.full((npad,), NP, jnp.int32)]).reshape(NCHUNK_PAD, CH)
    dstD = jnp.concatenate(
        [dst, jnp.full((npad,), NP, jnp.int32)]).reshape(NCHUNK_PAD, CH)
    edges_hist = jnp.stack([srcD, dstD])
    src2 = jnp.stack([src0, src0 + NP])

    zeros128 = jnp.zeros((NP, 128), jnp.float32)
    ones128 = jnp.ones((CH, 128), jnp.float32)

    deg16 = _sc_hist(edges_hist, ones128, zeros128)
    aux = _tc_aux(deg16)
    a16 = aux[0]
    b16 = aux[1]
    nn16 = aux[2]
    ind16 = aux[3]

    b1r = b1.reshape(1, 256)
    b2r = b2.reshape(1, 256)
    b3r = b3.reshape(1, 64)
    g1r = gamma1.reshape(1, 256)
    be1r = beta1.reshape(1, 256)
    g2r = gamma2.reshape(1, 256)
    be2r = beta2.reshape(1, 256)
    W3a = W3[:256]
    W3b = W3[256:]

    xp = jnp.pad(x, ((0, NP - N), (0, 0)))
    p1 = _tc_lin1(xp, W1, b1r, a16).reshape(2 * NP, 128)
    agg1 = _sc_spmm256(p1, src2, dstD, zeros128)
    p2 = _tc_bn_lin(agg1, b16, g1r, be1r, W2, b2r, a16).reshape(2 * NP, 128)
    agg2 = _sc_spmm256(p2, src2, dstD, zeros128)
    hainit = _tc_bn_lin3(agg2, b16, g2r, be2r, W3a, W3b, b3r, ind16)
    aggD = _sc_spmm64(hainit, src0, dstD, zeros128)
    nnagg = _tc_final(aggD, hainit, nn16)
    out128 = _sc_gather64(nnagg, src0)
    return out128[:, :64]


# speedup vs baseline: 9.4351x; 1.9613x over previous
import functools

import jax
import jax.numpy as jnp
from jax import lax
from jax.experimental import pallas as pl
from jax.experimental.pallas import tpu as pltpu
from jax.experimental.pallas import tpu_sc as plsc

N = 10000
NP = 10240
E = 160000
CH = 128
NCHUNK_PAD = 1280
EPAD = NCHUNK_PAD * CH
ROWS_PER_TILE = NP // 16

_MESH = plsc.VectorSubcoreMesh(
    core_axis_name="c", subcore_axis_name="s", num_cores=2, num_subcores=16)


NCHUNK = E // CH


def _pipelined_sweep(tbl, src_loc, n, maxn, rows0, rows1, sem0, sem1,
                     consume0, consume1):
    pltpu.async_copy(tbl.at[src_loc.at[0]], rows0, sem0)

    def body(k, carry):
        c0 = 2 * k
        c1 = c0 + 1

        @pl.when(c0 < n)
        def _():
            pltpu.make_async_copy(tbl.at[src_loc.at[c0]], rows0, sem0).wait()

            @pl.when(c1 < n)
            def _():
                pltpu.async_copy(tbl.at[src_loc.at[c1]], rows1, sem1)

            consume0(c0)

            @pl.when(c1 < n)
            def _():
                pltpu.make_async_copy(tbl.at[src_loc.at[c1]], rows1,
                                      sem1).wait()

                @pl.when(c1 + 1 < n)
                def _():
                    pltpu.async_copy(tbl.at[src_loc.at[c1 + 1]], rows0, sem0)

                consume1(c1)

        return carry

    lax.fori_loop(0, (maxn + 1) // 2, body, 0)


@functools.partial(
    pl.kernel,
    out_type=jax.ShapeDtypeStruct((2, NP, 128), jnp.float32),
    mesh=_MESH,
    scratch_types=[
        pltpu.VMEM((80, CH), jnp.int32),
        pltpu.VMEM((CH, 128), jnp.float32),
        pltpu.VMEM_SHARED((NP + 1, 128), jnp.float32),
    ],
)
def _sc_hist(edges, ones_h, zeros_h, out, idx_loc, ones_v, acc):
    c = lax.axis_index("c")
    s = lax.axis_index("s")
    r0 = s * ROWS_PER_TILE
    pltpu.sync_copy(zeros_h.at[pl.ds(r0, ROWS_PER_TILE)],
                    acc.at[pl.ds(r0, ROWS_PER_TILE)])
    pltpu.sync_copy(ones_h, ones_v)
    pltpu.sync_copy(edges.at[c, pl.ds(s * 80, 80)], idx_loc)
    n = jnp.minimum(80, NCHUNK - s * 80)
    plsc.subcore_barrier()

    def chunk(j, carry):
        pltpu.sync_copy(ones_v, acc.at[idx_loc.at[j]], add=True)
        return carry

    lax.fori_loop(0, n, chunk, 0)
    plsc.subcore_barrier()
    pltpu.sync_copy(acc.at[pl.ds(r0, ROWS_PER_TILE)],
                    out.at[c, pl.ds(r0, ROWS_PER_TILE)])


@functools.partial(
    pl.kernel,
    out_type=jax.ShapeDtypeStruct((2, NP, 128), jnp.float32),
    mesh=_MESH,
    scratch_types=[
        pltpu.VMEM((40, CH), jnp.int32),
        pltpu.VMEM((40, CH), jnp.int32),
        pltpu.VMEM((CH, 128), jnp.float32),
        pltpu.VMEM((CH, 128), jnp.float32),
        pltpu.SemaphoreType.DMA,
        pltpu.SemaphoreType.DMA,
        pltpu.VMEM_SHARED((NP + 1, 128), jnp.float32),
    ],
)
def _sc_spmm256(pflat, src2, dstp, zeros_h, out, src_loc, dst_loc, rows0,
                rows1, sem0, sem1, acc):
    c = lax.axis_index("c")
    s = lax.axis_index("s")
    r0 = s * ROWS_PER_TILE
    pltpu.sync_copy(zeros_h.at[pl.ds(r0, ROWS_PER_TILE)],
                    acc.at[pl.ds(r0, ROWS_PER_TILE)])
    plsc.subcore_barrier()

    def sc0(j):
        pltpu.sync_copy(rows0, acc.at[dst_loc.at[j]], add=True)

    def sc1(j):
        pltpu.sync_copy(rows1, acc.at[dst_loc.at[j]], add=True)

    for half in range(2):
        t0 = s * 80 + half * 40
        pltpu.sync_copy(src2.at[c, pl.ds(t0, 40)], src_loc)
        pltpu.sync_copy(dstp.at[pl.ds(t0, 40)], dst_loc)
        n = jnp.minimum(40, NCHUNK - t0)
        _pipelined_sweep(pflat, src_loc, n, 40, rows0, rows1, sem0, sem1,
                         sc0, sc1)
    plsc.subcore_barrier()
    pltpu.sync_copy(acc.at[pl.ds(r0, ROWS_PER_TILE)],
                    out.at[c, pl.ds(r0, ROWS_PER_TILE)])


@functools.partial(
    pl.kernel,
    out_type=jax.ShapeDtypeStruct((2, NP, 128), jnp.float32),
    mesh=_MESH,
    scratch_types=[
        pltpu.VMEM((40, CH), jnp.int32),
        pltpu.VMEM((40, CH), jnp.int32),
        pltpu.VMEM((CH, 128), jnp.float32),
        pltpu.VMEM((CH, 128), jnp.float32),
        pltpu.SemaphoreType.DMA,
        pltpu.SemaphoreType.DMA,
        pltpu.VMEM_SHARED((NP + 1, 128), jnp.float32),
    ],
)
def _sc_spmm64(table, srcp, dstp, zeros_h, out, src_loc, dst_loc, rows0,
               rows1, sem0, sem1, acc):
    c = lax.axis_index("c")
    s = lax.axis_index("s")
    r0 = s * ROWS_PER_TILE
    base = c * 640 + s * 40
    pltpu.sync_copy(zeros_h.at[pl.ds(r0, ROWS_PER_TILE)],
                    acc.at[pl.ds(r0, ROWS_PER_TILE)])
    pltpu.sync_copy(srcp.at[pl.ds(base, 40)], src_loc)
    pltpu.sync_copy(dstp.at[pl.ds(base, 40)], dst_loc)
    n = jnp.minimum(40, NCHUNK - base)
    plsc.subcore_barrier()

    def sc0(j):
        pltpu.sync_copy(rows0, acc.at[dst_loc.at[j]], add=True)

    def sc1(j):
        pltpu.sync_copy(rows1, acc.at[dst_loc.at[j]], add=True)

    _pipelined_sweep(table, src_loc, n, 40, rows0, rows1, sem0, sem1, sc0, sc1)
    plsc.subcore_barrier()
    pltpu.sync_copy(acc.at[pl.ds(r0, ROWS_PER_TILE)],
                    out.at[c, pl.ds(r0, ROWS_PER_TILE)])


@functools.partial(
    pl.kernel,
    out_type=jax.ShapeDtypeStruct((E, 128), jnp.float32),
    mesh=_MESH,
    scratch_types=[
        pltpu.VMEM((40, CH), jnp.int32),
        pltpu.VMEM((CH, 128), jnp.float32),
        pltpu.VMEM((CH, 128), jnp.float32),
        pltpu.SemaphoreType.DMA,
        pltpu.SemaphoreType.DMA,
    ],
)
def _sc_gather64(table, srcp, out, src_loc, rows0, rows1, sem0, sem1):
    c = lax.axis_index("c")
    s = lax.axis_index("s")
    base = c * 640 + s * 40
    pltpu.sync_copy(srcp.at[pl.ds(base, 40)], src_loc)
    n = jnp.minimum(40, NCHUNK - base)

    def st0(j):
        pltpu.sync_copy(rows0, out.at[pl.ds((base + j) * CH, CH)])

    def st1(j):
        pltpu.sync_copy(rows1, out.at[pl.ds((base + j) * CH, CH)])

    _pipelined_sweep(table, src_loc, n, 40, rows0, rows1, sem0, sem1, st0, st1)


def _tc_aux(deg16):
    def body(d_ref, o_ref):
        dout = d_ref[0][:, :16]
        din = d_ref[1][:, :16]
        a = lax.rsqrt(jnp.maximum(dout, 1.0))
        b = lax.rsqrt(jnp.maximum(din, 1.0))
        o_ref[0] = a
        o_ref[1] = b
        o_ref[2] = a * b
        o_ref[3] = din

    return pl.pallas_call(
        body, out_shape=jax.ShapeDtypeStruct((4, NP, 16), jnp.float32),
    )(deg16)


def _tc_lin1(x, W1, b1r, a16):
    R = 512

    def body(x_ref, w_ref, b_ref, a_ref, o_ref):
        h = jnp.dot(x_ref[...], w_ref[...],
                    preferred_element_type=jnp.float32) + b_ref[...]
        ph = h * a_ref[:, 0:1]
        o_ref[0] = ph[:, :128]
        o_ref[1] = ph[:, 128:]

    return pl.pallas_call(
        body,
        grid=(NP // R,),
        in_specs=[
            pl.BlockSpec((R, 128), lambda r: (r, 0)),
            pl.BlockSpec((128, 256), lambda r: (0, 0)),
            pl.BlockSpec((1, 256), lambda r: (0, 0)),
            pl.BlockSpec((R, 16), lambda r: (r, 0)),
        ],
        out_specs=pl.BlockSpec((2, R, 128), lambda r: (0, r, 0)),
        out_shape=jax.ShapeDtypeStruct((2, NP, 128), jnp.float32),
    )(x, W1, b1r, a16)


def _tc_bn_lin(agg, b16, g_r, be_r, W2, b2r, a16):
    R = 512

    def body(agg_ref, b_ref, g_ref, be_ref, w_ref, b2_ref, a_ref, o_ref,
             st_ref):
        p = pl.program_id(0)
        r = pl.program_id(1)
        o = jnp.concatenate([agg_ref[0], agg_ref[1]], axis=1) * b_ref[:, 0:1]

        @pl.when(jnp.logical_and(p == 0, r == 0))
        def _():
            st_ref[...] = jnp.zeros_like(st_ref)

        @pl.when(p == 0)
        def _():
            st_ref[0:1, :] += jnp.sum(o, axis=0, keepdims=True)
            st_ref[1:2, :] += jnp.sum(o * o, axis=0, keepdims=True)

        @pl.when(p == 1)
        def _():
            mu = st_ref[0:1, :] / float(N)
            var = st_ref[1:2, :] / float(N) - mu * mu
            z = (o - mu) * lax.rsqrt(var + 1e-5) * g_ref[...] + be_ref[...]
            z = jnp.maximum(z, 0.0)
            ph = (jnp.dot(z, w_ref[...], preferred_element_type=jnp.float32)
                  + b2_ref[...]) * a_ref[:, 0:1]
            o_ref[0] = ph[:, :128]
            o_ref[1] = ph[:, 128:]

    return pl.pallas_call(
        body,
        grid=(2, NP // R),
        in_specs=[
            pl.BlockSpec((2, R, 128), lambda p, r: (0, r, 0)),
            pl.BlockSpec((R, 16), lambda p, r: (r, 0)),
            pl.BlockSpec((1, 256), lambda p, r: (0, 0)),
            pl.BlockSpec((1, 256), lambda p, r: (0, 0)),
            pl.BlockSpec((256, 256), lambda p, r: (0, 0)),
            pl.BlockSpec((1, 256), lambda p, r: (0, 0)),
            pl.BlockSpec((R, 16), lambda p, r: (r, 0)),
        ],
        out_specs=pl.BlockSpec((2, R, 128), lambda p, r: (0, r, 0)),
        out_shape=jax.ShapeDtypeStruct((2, NP, 128), jnp.float32),
        scratch_shapes=[pltpu.VMEM((2, 256), jnp.float32)],
    )(agg, b16, g_r, be_r, W2, b2r, a16)


def _tc_bn_lin3(agg, b16, g_r, be_r, W3a, W3b, b3r, ind16):
    R = 512

    def body(agg_ref, b_ref, g_ref, be_ref, wa_ref, wb_ref, b3_ref, ind_ref,
             o_ref, st_ref):
        p = pl.program_id(0)
        r = pl.program_id(1)
        o = jnp.concatenate([agg_ref[0], agg_ref[1]], axis=1) * b_ref[:, 0:1]

        @pl.when(jnp.logical_and(p == 0, r == 0))
        def _():
            st_ref[...] = jnp.zeros_like(st_ref)

        @pl.when(p == 0)
        def _():
            st_ref[0:1, :] += jnp.sum(o, axis=0, keepdims=True)
            st_ref[1:2, :] += jnp.sum(o * o, axis=0, keepdims=True)

        @pl.when(p == 1)
        def _():
            mu = st_ref[0:1, :] / float(N)
            var = st_ref[1:2, :] / float(N) - mu * mu
            z = (o - mu) * lax.rsqrt(var + 1e-5) * g_ref[...] + be_ref[...]
            z = jnp.maximum(z, 0.0)
            ha = jnp.dot(z, wa_ref[...], preferred_element_type=jnp.float32)
            init = ind_ref[:, 0:1] * (
                jnp.dot(z, wb_ref[...], preferred_element_type=jnp.float32)
                + b3_ref[...])
            o_ref[...] = jnp.concatenate([ha, init], axis=1)

    return pl.pallas_call(
        body,
        grid=(2, NP // R),
        in_specs=[
            pl.BlockSpec((2, R, 128), lambda p, r: (0, r, 0)),
            pl.BlockSpec((R, 16), lambda p, r: (r, 0)),
            pl.BlockSpec((1, 256), lambda p, r: (0, 0)),
            pl.BlockSpec((1, 256), lambda p, r: (0, 0)),
            pl.BlockSpec((256, 64), lambda p, r: (0, 0)),
            pl.BlockSpec((256, 64), lambda p, r: (0, 0)),
            pl.BlockSpec((1, 64), lambda p, r: (0, 0)),
            pl.BlockSpec((R, 16), lambda p, r: (r, 0)),
        ],
        out_specs=pl.BlockSpec((R, 128), lambda p, r: (r, 0)),
        out_shape=jax.ShapeDtypeStruct((NP, 128), jnp.float32),
        scratch_shapes=[pltpu.VMEM((2, 256), jnp.float32)],
    )(agg, b16, g_r, be_r, W3a, W3b, b3r, ind16)


def _tc_final(aggD, hainit, nn16):
    def body(agg_ref, hi_ref, nn_ref, o_ref):
        left = nn_ref[:, 0:1] * (agg_ref[0][:, :64] + agg_ref[1][:, :64]
                                 + hi_ref[:, 64:])
        o_ref[...] = jnp.concatenate([left, jnp.zeros_like(left)], axis=1)

    return pl.pallas_call(
        body, out_shape=jax.ShapeDtypeStruct((NP, 128), jnp.float32),
    )(aggD, hainit, nn16)


def kernel(x, edge_index, W1, b1, gamma1, beta1, W2, b2, gamma2, beta2, W3,
           b3):
    src = edge_index[0]
    dst = edge_index[1]
    npad = EPAD - E
    src0 = jnp.concatenate(
        [src, jnp.zeros((npad,), jnp.int32)]).reshape(NCHUNK_PAD, CH)
    srcD = jnp.concatenate(
        [src, jnp.full((npad,), NP, jnp.int32)]).reshape(NCHUNK_PAD, CH)
    dstD = jnp.concatenate(
        [dst, jnp.full((npad,), NP, jnp.int32)]).reshape(NCHUNK_PAD, CH)
    edges_hist = jnp.stack([srcD, dstD])
    src2 = jnp.stack([src0, src0 + NP])

    zeros128 = jnp.zeros((NP, 128), jnp.float32)
    ones128 = jnp.ones((CH, 128), jnp.float32)

    deg16 = _sc_hist(edges_hist, ones128, zeros128)
    aux = _tc_aux(deg16)
    a16 = aux[0]
    b16 = aux[1]
    nn16 = aux[2]
    ind16 = aux[3]

    b1r = b1.reshape(1, 256)
    b2r = b2.reshape(1, 256)
    b3r = b3.reshape(1, 64)
    g1r = gamma1.reshape(1, 256)
    be1r = beta1.reshape(1, 256)
    g2r = gamma2.reshape(1, 256)
    be2r = beta2.reshape(1, 256)
    W3a = W3[:256]
    W3b = W3[256:]

    xp = jnp.pad(x, ((0, NP - N), (0, 0)))
    p1 = _tc_lin1(xp, W1, b1r, a16).reshape(2 * NP, 128)
    agg1 = _sc_spmm256(p1, src2, dstD, zeros128)
    p2 = _tc_bn_lin(agg1, b16, g1r, be1r, W2, b2r, a16).reshape(2 * NP, 128)
    agg2 = _sc_spmm256(p2, src2, dstD, zeros128)
    hainit = _tc_bn_lin3(agg2, b16, g2r, be2r, W3a, W3b, b3r, ind16)
    aggD = _sc_spmm64(hainit, src0, dstD, zeros128)
    nnagg = _tc_final(aggD, hainit, nn16)
    out128 = _sc_gather64(nnagg, src0)
    return out128[:, :64]


# speedup vs baseline: 9.7880x; 1.0374x over previous
import functools

import jax
import jax.numpy as jnp
from jax import lax
from jax.experimental import pallas as pl
from jax.experimental.pallas import tpu as pltpu
from jax.experimental.pallas import tpu_sc as plsc

N = 10000
NP = 10240
E = 160000
CH = 128
NCHUNK_PAD = 1280
EPAD = NCHUNK_PAD * CH
ROWS_PER_TILE = NP // 16

_MESH = plsc.VectorSubcoreMesh(
    core_axis_name="c", subcore_axis_name="s", num_cores=2, num_subcores=16)


NCHUNK = E // CH


def _pipelined_sweep(tbl, src_loc, n, maxn, rows0, rows1, sem0, sem1,
                     consume0, consume1):
    pltpu.async_copy(tbl.at[src_loc.at[0]], rows0, sem0)

    def body(k, carry):
        c0 = 2 * k
        c1 = c0 + 1

        @pl.when(c0 < n)
        def _():
            pltpu.make_async_copy(tbl.at[src_loc.at[c0]], rows0, sem0).wait()

            @pl.when(c1 < n)
            def _():
                pltpu.async_copy(tbl.at[src_loc.at[c1]], rows1, sem1)

            consume0(c0)

            @pl.when(c1 < n)
            def _():
                pltpu.make_async_copy(tbl.at[src_loc.at[c1]], rows1,
                                      sem1).wait()

                @pl.when(c1 + 1 < n)
                def _():
                    pltpu.async_copy(tbl.at[src_loc.at[c1 + 1]], rows0, sem0)

                consume1(c1)

        return carry

    lax.fori_loop(0, (maxn + 1) // 2, body, 0)


@functools.partial(
    pl.kernel,
    out_type=jax.ShapeDtypeStruct((2, NP, 128), jnp.float32),
    mesh=_MESH,
    scratch_types=[
        pltpu.VMEM((80, CH), jnp.int32),
        pltpu.VMEM((CH, 128), jnp.float32),
        pltpu.VMEM_SHARED((NP + 1, 128), jnp.float32),
    ],
)
def _sc_hist(edges, ones_h, zeros_h, out, idx_loc, ones_v, acc):
    c = lax.axis_index("c")
    s = lax.axis_index("s")
    r0 = s * ROWS_PER_TILE
    pltpu.sync_copy(zeros_h.at[pl.ds(r0, ROWS_PER_TILE)],
                    acc.at[pl.ds(r0, ROWS_PER_TILE)])
    pltpu.sync_copy(ones_h, ones_v)
    pltpu.sync_copy(edges.at[c, pl.ds(s * 80, 80)], idx_loc)
    n = jnp.minimum(80, NCHUNK - s * 80)
    plsc.subcore_barrier()

    def chunk(j, carry):
        pltpu.sync_copy(ones_v, acc.at[idx_loc.at[j]], add=True)
        return carry

    lax.fori_loop(0, n, chunk, 0)
    plsc.subcore_barrier()
    pltpu.sync_copy(acc.at[pl.ds(r0, ROWS_PER_TILE)],
                    out.at[c, pl.ds(r0, ROWS_PER_TILE)])


@functools.partial(
    pl.kernel,
    out_type=jax.ShapeDtypeStruct((2, NP, 128), jnp.float32),
    mesh=_MESH,
    scratch_types=[
        pltpu.VMEM((40, CH), jnp.int32),
        pltpu.VMEM((40, CH), jnp.int32),
        pltpu.VMEM((CH, 128), jnp.float32),
        pltpu.VMEM((CH, 128), jnp.float32),
        pltpu.SemaphoreType.DMA,
        pltpu.SemaphoreType.DMA,
        pltpu.VMEM_SHARED((NP + 1, 128), jnp.float32),
    ],
)
def _sc_spmm256(pflat, src2, dstp, zeros_h, out, src_loc, dst_loc, rows0,
                rows1, sem0, sem1, acc):
    c = lax.axis_index("c")
    s = lax.axis_index("s")
    r0 = s * ROWS_PER_TILE
    pltpu.sync_copy(zeros_h.at[pl.ds(r0, ROWS_PER_TILE)],
                    acc.at[pl.ds(r0, ROWS_PER_TILE)])
    plsc.subcore_barrier()

    def sc0(j):
        pltpu.sync_copy(rows0, acc.at[dst_loc.at[j]], add=True)

    def sc1(j):
        pltpu.sync_copy(rows1, acc.at[dst_loc.at[j]], add=True)

    for half in range(2):
        t0 = s * 80 + half * 40
        pltpu.sync_copy(src2.at[c, pl.ds(t0, 40)], src_loc)
        pltpu.sync_copy(dstp.at[pl.ds(t0, 40)], dst_loc)
        n = jnp.minimum(40, NCHUNK - t0)
        _pipelined_sweep(pflat, src_loc, n, 40, rows0, rows1, sem0, sem1,
                         sc0, sc1)
    plsc.subcore_barrier()
    pltpu.sync_copy(acc.at[pl.ds(r0, ROWS_PER_TILE)],
                    out.at[c, pl.ds(r0, ROWS_PER_TILE)])


@functools.partial(
    pl.kernel,
    out_type=jax.ShapeDtypeStruct((2, NP, 128), jnp.float32),
    mesh=_MESH,
    scratch_types=[
        pltpu.VMEM((40, CH), jnp.int32),
        pltpu.VMEM((40, CH), jnp.int32),
        pltpu.VMEM((CH, 128), jnp.float32),
        pltpu.VMEM((CH, 128), jnp.float32),
        pltpu.SemaphoreType.DMA,
        pltpu.SemaphoreType.DMA,
        pltpu.VMEM_SHARED((NP + 1, 128), jnp.float32),
    ],
)
def _sc_spmm64(table, srcp, dstp, zeros_h, out, src_loc, dst_loc, rows0,
               rows1, sem0, sem1, acc):
    c = lax.axis_index("c")
    s = lax.axis_index("s")
    r0 = s * ROWS_PER_TILE
    base = c * 640 + s * 40
    pltpu.sync_copy(zeros_h.at[pl.ds(r0, ROWS_PER_TILE)],
                    acc.at[pl.ds(r0, ROWS_PER_TILE)])
    pltpu.sync_copy(srcp.at[pl.ds(base, 40)], src_loc)
    pltpu.sync_copy(dstp.at[pl.ds(base, 40)], dst_loc)
    n = jnp.minimum(40, NCHUNK - base)
    plsc.subcore_barrier()

    def sc0(j):
        pltpu.sync_copy(rows0, acc.at[dst_loc.at[j]], add=True)

    def sc1(j):
        pltpu.sync_copy(rows1, acc.at[dst_loc.at[j]], add=True)

    _pipelined_sweep(table, src_loc, n, 40, rows0, rows1, sem0, sem1, sc0, sc1)
    plsc.subcore_barrier()
    pltpu.sync_copy(acc.at[pl.ds(r0, ROWS_PER_TILE)],
                    out.at[c, pl.ds(r0, ROWS_PER_TILE)])


@functools.partial(
    pl.kernel,
    out_type=jax.ShapeDtypeStruct((E, 128), jnp.float32),
    mesh=_MESH,
    scratch_types=[
        pltpu.VMEM((40, CH), jnp.int32),
        pltpu.VMEM((CH, 128), jnp.float32),
        pltpu.VMEM((CH, 128), jnp.float32),
        pltpu.SemaphoreType.DMA,
        pltpu.SemaphoreType.DMA,
    ],
)
def _sc_gather64(table, srcp, out, src_loc, rows0, rows1, sem0, sem1):
    c = lax.axis_index("c")
    s = lax.axis_index("s")
    base = c * 640 + s * 40
    pltpu.sync_copy(srcp.at[pl.ds(base, 40)], src_loc)
    n = jnp.minimum(40, NCHUNK - base)

    def st0(j):
        pltpu.sync_copy(rows0, out.at[pl.ds((base + j) * CH, CH)])

    def st1(j):
        pltpu.sync_copy(rows1, out.at[pl.ds((base + j) * CH, CH)])

    _pipelined_sweep(table, src_loc, n, 40, rows0, rows1, sem0, sem1, st0, st1)


def _tc_lin1(x, W1, b1r, deg16):
    R = 512

    def body(x_ref, w_ref, b_ref, d_ref, o_ref):
        a = lax.rsqrt(jnp.maximum(d_ref[0][:, 0:1], 1.0))
        h = jnp.dot(x_ref[...], w_ref[...],
                    preferred_element_type=jnp.float32) + b_ref[...]
        ph = h * a
        o_ref[0] = ph[:, :128]
        o_ref[1] = ph[:, 128:]

    return pl.pallas_call(
        body,
        grid=(NP // R,),
        in_specs=[
            pl.BlockSpec((R, 128), lambda r: (r, 0)),
            pl.BlockSpec((128, 256), lambda r: (0, 0)),
            pl.BlockSpec((1, 256), lambda r: (0, 0)),
            pl.BlockSpec((2, R, 128), lambda r: (0, r, 0)),
        ],
        out_specs=pl.BlockSpec((2, R, 128), lambda r: (0, r, 0)),
        out_shape=jax.ShapeDtypeStruct((2, NP, 128), jnp.float32),
    )(x, W1, b1r, deg16)


def _tc_bn_lin(agg, deg16, g_r, be_r, W2, b2r):
    R = 512

    def body(agg_ref, d_ref, g_ref, be_ref, w_ref, b2_ref, o_ref,
             st_ref):
        p = pl.program_id(0)
        r = pl.program_id(1)
        b = lax.rsqrt(jnp.maximum(d_ref[1][:, 0:1], 1.0))
        o = jnp.concatenate([agg_ref[0], agg_ref[1]], axis=1) * b

        @pl.when(jnp.logical_and(p == 0, r == 0))
        def _():
            st_ref[...] = jnp.zeros_like(st_ref)

        @pl.when(p == 0)
        def _():
            st_ref[0:1, :] += jnp.sum(o, axis=0, keepdims=True)
            st_ref[1:2, :] += jnp.sum(o * o, axis=0, keepdims=True)

        @pl.when(p == 1)
        def _():
            mu = st_ref[0:1, :] / float(N)
            var = st_ref[1:2, :] / float(N) - mu * mu
            z = (o - mu) * lax.rsqrt(var + 1e-5) * g_ref[...] + be_ref[...]
            z = jnp.maximum(z, 0.0)
            a = lax.rsqrt(jnp.maximum(d_ref[0][:, 0:1], 1.0))
            ph = (jnp.dot(z, w_ref[...], preferred_element_type=jnp.float32)
                  + b2_ref[...]) * a
            o_ref[0] = ph[:, :128]
            o_ref[1] = ph[:, 128:]

    return pl.pallas_call(
        body,
        grid=(2, NP // R),
        in_specs=[
            pl.BlockSpec((2, R, 128), lambda p, r: (0, r, 0)),
            pl.BlockSpec((2, R, 128), lambda p, r: (0, r, 0)),
            pl.BlockSpec((1, 256), lambda p, r: (0, 0)),
            pl.BlockSpec((1, 256), lambda p, r: (0, 0)),
            pl.BlockSpec((256, 256), lambda p, r: (0, 0)),
            pl.BlockSpec((1, 256), lambda p, r: (0, 0)),
        ],
        out_specs=pl.BlockSpec((2, R, 128), lambda p, r: (0, r, 0)),
        out_shape=jax.ShapeDtypeStruct((2, NP, 128), jnp.float32),
        scratch_shapes=[pltpu.VMEM((2, 256), jnp.float32)],
    )(agg, deg16, g_r, be_r, W2, b2r)


def _tc_bn_lin3(agg, deg16, g_r, be_r, W3a, W3b, b3r):
    R = 512

    def body(agg_ref, d_ref, g_ref, be_ref, wa_ref, wb_ref, b3_ref,
             o_ref, st_ref):
        p = pl.program_id(0)
        r = pl.program_id(1)
        b = lax.rsqrt(jnp.maximum(d_ref[1][:, 0:1], 1.0))
        o = jnp.concatenate([agg_ref[0], agg_ref[1]], axis=1) * b

        @pl.when(jnp.logical_and(p == 0, r == 0))
        def _():
            st_ref[...] = jnp.zeros_like(st_ref)

        @pl.when(p == 0)
        def _():
            st_ref[0:1, :] += jnp.sum(o, axis=0, keepdims=True)
            st_ref[1:2, :] += jnp.sum(o * o, axis=0, keepdims=True)

        @pl.when(p == 1)
        def _():
            mu = st_ref[0:1, :] / float(N)
            var = st_ref[1:2, :] / float(N) - mu * mu
            z = (o - mu) * lax.rsqrt(var + 1e-5) * g_ref[...] + be_ref[...]
            z = jnp.maximum(z, 0.0)
            ha = jnp.dot(z, wa_ref[...], preferred_element_type=jnp.float32)
            init = d_ref[1][:, 0:1] * (
                jnp.dot(z, wb_ref[...], preferred_element_type=jnp.float32)
                + b3_ref[...])
            o_ref[...] = jnp.concatenate([ha, init], axis=1)

    return pl.pallas_call(
        body,
        grid=(2, NP // R),
        in_specs=[
            pl.BlockSpec((2, R, 128), lambda p, r: (0, r, 0)),
            pl.BlockSpec((2, R, 128), lambda p, r: (0, r, 0)),
            pl.BlockSpec((1, 256), lambda p, r: (0, 0)),
            pl.BlockSpec((1, 256), lambda p, r: (0, 0)),
            pl.BlockSpec((256, 64), lambda p, r: (0, 0)),
            pl.BlockSpec((256, 64), lambda p, r: (0, 0)),
            pl.BlockSpec((1, 64), lambda p, r: (0, 0)),
        ],
        out_specs=pl.BlockSpec((R, 128), lambda p, r: (r, 0)),
        out_shape=jax.ShapeDtypeStruct((NP, 128), jnp.float32),
        scratch_shapes=[pltpu.VMEM((2, 256), jnp.float32)],
    )(agg, deg16, g_r, be_r, W3a, W3b, b3r)


def _tc_final(aggD, hainit, deg16):
    def body(agg_ref, hi_ref, d_ref, o_ref):
        nn = lax.rsqrt(jnp.maximum(d_ref[0][:, 0:1], 1.0)
                       * jnp.maximum(d_ref[1][:, 0:1], 1.0))
        left = nn * (agg_ref[0][:, :64] + agg_ref[1][:, :64]
                     + hi_ref[:, 64:])
        o_ref[...] = jnp.concatenate([left, jnp.zeros_like(left)], axis=1)

    return pl.pallas_call(
        body, out_shape=jax.ShapeDtypeStruct((NP, 128), jnp.float32),
    )(aggD, hainit, deg16)


def kernel(x, edge_index, W1, b1, gamma1, beta1, W2, b2, gamma2, beta2, W3,
           b3):
    src = edge_index[0]
    dst = edge_index[1]
    npad = EPAD - E
    src0 = jnp.concatenate(
        [src, jnp.zeros((npad,), jnp.int32)]).reshape(NCHUNK_PAD, CH)
    srcD = jnp.concatenate(
        [src, jnp.full((npad,), NP, jnp.int32)]).reshape(NCHUNK_PAD, CH)
    dstD = jnp.concatenate(
        [dst, jnp.full((npad,), NP, jnp.int32)]).reshape(NCHUNK_PAD, CH)
    edges_hist = jnp.stack([srcD, dstD])
    src2 = jnp.stack([src0, src0 + NP])

    zeros128 = jnp.zeros((NP, 128), jnp.float32)
    ones128 = jnp.ones((CH, 128), jnp.float32)

    deg16 = _sc_hist(edges_hist, ones128, zeros128)

    b1r = b1.reshape(1, 256)
    b2r = b2.reshape(1, 256)
    b3r = b3.reshape(1, 64)
    g1r = gamma1.reshape(1, 256)
    be1r = beta1.reshape(1, 256)
    g2r = gamma2.reshape(1, 256)
    be2r = beta2.reshape(1, 256)
    W3a = W3[:256]
    W3b = W3[256:]

    xp = jnp.pad(x, ((0, NP - N), (0, 0)))
    p1 = _tc_lin1(xp, W1, b1r, deg16).reshape(2 * NP, 128)
    agg1 = _sc_spmm256(p1, src2, dstD, zeros128)
    p2 = _tc_bn_lin(agg1, deg16, g1r, be1r, W2, b2r).reshape(2 * NP, 128)
    agg2 = _sc_spmm256(p2, src2, dstD, zeros128)
    hainit = _tc_bn_lin3(agg2, deg16, g2r, be2r, W3a, W3b, b3r)
    aggD = _sc_spmm64(hainit, src0, dstD, zeros128)
    nnagg = _tc_final(aggD, hainit, deg16)
    out128 = _sc_gather64(nnagg, src0)
    return out128[:, :64]
